# baseline (speedup 1.0000x reference)
import functools

import jax
import jax.numpy as jnp
from jax import lax
from jax.experimental import pallas as pl
from jax.experimental.pallas import tpu as pltpu
from jax.experimental.pallas import tpu_sc as plsc

BATCH = 16384
EMBED_DIM = 32

try:
    _info = plsc.get_sparse_core_info()
    _NC = _info.num_cores
    _NS = _info.num_subcores
except Exception:
    _NC, _NS = 2, 16
_NW = _NC * _NS
_BPW = BATCH // _NW
_CHUNK = 128
_NCHUNK = _BPW // _CHUNK

_MV_BLK = 65536


def _mv2_body(u_ref, m_ref, wu_ref, wm_ref, ou_ref, om_ref):
    i = pl.program_id(0)
    s = pl.program_id(1)
    nu = pl.num_programs(0) - 1

    @pl.when(i < nu)
    def _():
        part = lax.dot_general(
            wu_ref[0], u_ref[0],
            dimension_numbers=(((0,), (0,)), ((), ())),
            preferred_element_type=jnp.float32,
        )[0]

        @pl.when(s == 0)
        def _():
            ou_ref[...] = part

        @pl.when(s > 0)
        def _():
            ou_ref[...] = ou_ref[...] + part

    @pl.when(jnp.logical_and(i == nu, s == 0))
    def _():
        om_ref[...] = lax.dot_general(
            wm_ref[...], m_ref[...],
            dimension_numbers=(((0,), (0,)), ((), ())),
            preferred_element_type=jnp.float32,
        )[0]


def _matvec2(ut_t, mt_t, wu, wm):
    d, nu = ut_t.shape
    _, nm = mt_t.shape
    gu = (nu + _MV_BLK - 1) // _MV_BLK
    mblk = ((nm + 1023) // 1024) * 1024
    ut4 = ut_t.reshape(4, 8, nu)
    wu4 = wu.reshape(4, 8, 1)
    return pl.pallas_call(
        _mv2_body,
        grid=(gu + 1, 4),
        in_specs=[
            pl.BlockSpec((1, 8, _MV_BLK),
                         lambda i, s: (s, 0, jnp.minimum(i, gu - 1))),
            pl.BlockSpec((d, mblk), lambda i, s: (0, 0)),
            pl.BlockSpec((1, 8, 1), lambda i, s: (s, 0, 0)),
            pl.BlockSpec((d, 1), lambda i, s: (0, 0)),
        ],
        out_specs=[
            pl.BlockSpec((_MV_BLK,), lambda i, s: (jnp.minimum(i, gu - 1),)),
            pl.BlockSpec((mblk,), lambda i, s: (0,)),
        ],
        out_shape=[
            jax.ShapeDtypeStruct((nu,), jnp.float32),
            jax.ShapeDtypeStruct((nm,), jnp.float32),
        ],
    )(ut4, mt_t, wu4, wm)


def _sc_body(uid_hbm, mid_hbm, udot_hbm, mdot_hbm, b_hbm, out_hbm,
             uidx, midx, uval, mval, bv, outv, sem):
    wid = lax.axis_index("s") * _NC + lax.axis_index("c")
    base = wid * _BPW

    pltpu.sync_copy(uid_hbm.at[wid], uidx)
    pltpu.sync_copy(mid_hbm.at[wid], midx)
    pltpu.sync_copy(b_hbm, bv)

    copies = []
    for c in range(_NCHUNK):
        copies.append(pltpu.async_copy(udot_hbm.at[uidx.at[c]], uval.at[c], sem))
        copies.append(pltpu.async_copy(mdot_hbm.at[midx.at[c]], mval.at[c], sem))
    for cp in copies:
        cp.wait()

    bvec = bv[...]
    for c in range(_NCHUNK):
        for k in range(_CHUNK // 16):
            v = uval[c, pl.ds(k * 16, 16)] + mval[c, pl.ds(k * 16, 16)] + bvec
            outv[pl.ds(c * _CHUNK + k * 16, 16)] = v

    pltpu.sync_copy(outv, out_hbm.at[pl.ds(base, _BPW)])


@jax.jit
def _run(user_ids, movie_ids, user_table, movie_table, fc_w, fc_b):
    udot, mdot = _matvec2(user_table.T, movie_table.T,
                          fc_w[:EMBED_DIM], fc_w[EMBED_DIM:])
    uid3d = user_ids.astype(jnp.int32).reshape(_NW, _NCHUNK, _CHUNK)
    mid3d = movie_ids.astype(jnp.int32).reshape(_NW, _NCHUNK, _CHUNK)
    bias16 = jnp.broadcast_to(fc_b.reshape(()), (16,))

    g = functools.partial(
        pl.kernel,
        mesh=plsc.VectorSubcoreMesh(core_axis_name="c", subcore_axis_name="s"),
        out_type=jax.ShapeDtypeStruct((BATCH,), jnp.float32),
        compiler_params=pltpu.CompilerParams(
            needs_layout_passes=False, use_tc_tiling_on_sc=False),
        scratch_types=[
            pltpu.VMEM((_NCHUNK, _CHUNK), jnp.int32),
            pltpu.VMEM((_NCHUNK, _CHUNK), jnp.int32),
            pltpu.VMEM((_NCHUNK, _CHUNK), jnp.float32),
            pltpu.VMEM((_NCHUNK, _CHUNK), jnp.float32),
            pltpu.VMEM((16,), jnp.float32),
            pltpu.VMEM((_BPW,), jnp.float32),
            pltpu.SemaphoreType.DMA,
        ],
    )(_sc_body)
    return g(uid3d, mid3d, udot, mdot, bias16)


def kernel(user_ids, movie_ids, user_table, movie_table, fc_w, fc_b):
    return _run(user_ids, movie_ids, user_table, movie_table, fc_w, fc_b)


# speedup vs baseline: 1.5266x; 1.5266x over previous
import functools

import jax
import jax.numpy as jnp
from jax import lax
from jax.experimental import pallas as pl
from jax.experimental.pallas import tpu as pltpu
from jax.experimental.pallas import tpu_sc as plsc

BATCH = 16384
EMBED_DIM = 32

try:
    _info = plsc.get_sparse_core_info()
    _NC = _info.num_cores
    _NS = _info.num_subcores
except Exception:
    _NC, _NS = 2, 16
_NW = _NC * _NS
_BPW = BATCH // _NW
_CHUNK = 128
_NCHUNK = _BPW // _CHUNK

_MV_BLK = 65536


def _mv2_body(u_ref, m_ref, wu_ref, wm_ref, ou_ref, om_ref):
    i = pl.program_id(0)
    nu = pl.num_programs(0) - 1

    @pl.when(i < nu)
    def _():
        ou_ref[...] = lax.dot_general(
            wu_ref[...], u_ref[...],
            dimension_numbers=(((0,), (0,)), ((), ())),
            preferred_element_type=jnp.float32,
        )[0]

    @pl.when(i == nu)
    def _():
        om_ref[...] = lax.dot_general(
            wm_ref[...], m_ref[...],
            dimension_numbers=(((0,), (0,)), ((), ())),
            preferred_element_type=jnp.float32,
        )[0]


def _matvec2(ut_t, mt_t, wu, wm):
    d, nu = ut_t.shape
    _, nm = mt_t.shape
    gu = (nu + _MV_BLK - 1) // _MV_BLK
    mblk = ((nm + 1023) // 1024) * 1024
    return pl.pallas_call(
        _mv2_body,
        grid=(gu + 1,),
        in_specs=[
            pl.BlockSpec((d, _MV_BLK), lambda i: (0, jnp.minimum(i, gu - 1))),
            pl.BlockSpec((d, mblk), lambda i: (0, 0)),
            pl.BlockSpec((d, 1), lambda i: (0, 0)),
            pl.BlockSpec((d, 1), lambda i: (0, 0)),
        ],
        out_specs=[
            pl.BlockSpec((_MV_BLK,), lambda i: (jnp.minimum(i, gu - 1),)),
            pl.BlockSpec((mblk,), lambda i: (0,)),
        ],
        out_shape=[
            jax.ShapeDtypeStruct((nu,), jnp.float32),
            jax.ShapeDtypeStruct((nm,), jnp.float32),
        ],
    )(ut_t, mt_t, wu, wm)


def _sc_body(uid_hbm, mid_hbm, udot_hbm, mdot_hbm, b_hbm, out_hbm,
             uidx, midx, uval, mval, bv, outv, sem):
    wid = lax.axis_index("s") * _NC + lax.axis_index("c")
    base = wid * _BPW

    pltpu.sync_copy(uid_hbm.at[wid], uidx)
    pltpu.sync_copy(mid_hbm.at[wid], midx)
    pltpu.sync_copy(b_hbm, bv)

    copies = []
    for c in range(_NCHUNK):
        copies.append(pltpu.async_copy(udot_hbm.at[uidx.at[c]], uval.at[c], sem))
        copies.append(pltpu.async_copy(mdot_hbm.at[midx.at[c]], mval.at[c], sem))
    for cp in copies:
        cp.wait()

    bvec = bv[...]
    for c in range(_NCHUNK):
        for k in range(_CHUNK // 16):
            v = uval[c, pl.ds(k * 16, 16)] + mval[c, pl.ds(k * 16, 16)] + bvec
            outv[pl.ds(c * _CHUNK + k * 16, 16)] = v

    pltpu.sync_copy(outv, out_hbm.at[pl.ds(base, _BPW)])


@jax.jit
def _run(user_ids, movie_ids, user_table, movie_table, fc_w, fc_b):
    udot, mdot = _matvec2(user_table.T, movie_table.T,
                          fc_w[:EMBED_DIM], fc_w[EMBED_DIM:])
    uid3d = user_ids.astype(jnp.int32).reshape(_NW, _NCHUNK, _CHUNK)
    mid3d = movie_ids.astype(jnp.int32).reshape(_NW, _NCHUNK, _CHUNK)
    bias16 = jnp.broadcast_to(fc_b.reshape(()), (16,))

    g = functools.partial(
        pl.kernel,
        mesh=plsc.VectorSubcoreMesh(core_axis_name="c", subcore_axis_name="s"),
        out_type=jax.ShapeDtypeStruct((BATCH,), jnp.float32),
        compiler_params=pltpu.CompilerParams(
            needs_layout_passes=False, use_tc_tiling_on_sc=False),
        scratch_types=[
            pltpu.VMEM((_NCHUNK, _CHUNK), jnp.int32),
            pltpu.VMEM((_NCHUNK, _CHUNK), jnp.int32),
            pltpu.VMEM((_NCHUNK, _CHUNK), jnp.float32),
            pltpu.VMEM((_NCHUNK, _CHUNK), jnp.float32),
            pltpu.VMEM((16,), jnp.float32),
            pltpu.VMEM((_BPW,), jnp.float32),
            pltpu.SemaphoreType.DMA,
        ],
    )(_sc_body)
    return g(uid3d, mid3d, udot, mdot, bias16)


def kernel(user_ids, movie_ids, user_table, movie_table, fc_w, fc_b):
    return _run(user_ids, movie_ids, user_table, movie_table, fc_w, fc_b)
